# baseline (speedup 1.0000x reference)
import functools

import jax
import jax.numpy as jnp
from jax import lax
from jax.experimental import pallas as pl
from jax.experimental.pallas import tpu as pltpu
from jax.experimental.pallas import tpu_sc as plsc

_N = 2048
_L = 16
_NC = 2
_NS = 16
_NW = _NC * _NS
_QPW = _N // _NW
_NCH = _QPW // _L
_PQ = 1

_STEPLIM = 5
_TOL = 1e-4


def _make_knn_kernel():
    mesh = plsc.VectorSubcoreMesh(
        core_axis_name="c", subcore_axis_name="s",
        num_cores=_NC, num_subcores=_NS)

    out_t = jax.ShapeDtypeStruct((_N,), jnp.float32)

    @functools.partial(
        pl.kernel,
        out_type=(out_t, out_t, out_t, out_t),
        mesh=mesh,
        compiler_params=pltpu.CompilerParams(needs_layout_passes=False),
        scratch_types=(
            pltpu.VMEM((_N,), jnp.float32),
            pltpu.VMEM((_N,), jnp.float32),
            pltpu.VMEM((_N,), jnp.float32),
            pltpu.VMEM((_QPW,), jnp.float32),
            pltpu.VMEM((_QPW,), jnp.float32),
            pltpu.VMEM((_QPW,), jnp.float32),
            pltpu.VMEM((_QPW,), jnp.float32),
            pltpu.VMEM((_QPW,), jnp.float32),
            pltpu.VMEM((_QPW,), jnp.float32),
            pltpu.VMEM((_QPW,), jnp.float32),
        ),
    )
    def knn(qx_hbm, qy_hbm, qz_hbm, tx_hbm, ty_hbm, tz_hbm,
            d2_hbm, mx_hbm, my_hbm, mz_hbm,
            tx_v, ty_v, tz_v, qx_v, qy_v, qz_v, o_d2, o_mx, o_my, o_mz):
        wid = lax.axis_index("s") * _NC + lax.axis_index("c")
        base = wid * _QPW

        pltpu.sync_copy(tx_hbm, tx_v)
        pltpu.sync_copy(ty_hbm, ty_v)
        pltpu.sync_copy(tz_hbm, tz_v)
        pltpu.sync_copy(qx_hbm.at[pl.ds(base, _QPW)], qx_v)
        pltpu.sync_copy(qy_hbm.at[pl.ds(base, _QPW)], qy_v)
        pltpu.sync_copy(qz_hbm.at[pl.ds(base, _QPW)], qz_v)

        big = jnp.full((_L,), jnp.inf, dtype=jnp.float32)
        zero = jnp.zeros((_L,), dtype=jnp.int32)

        mins = [None] * _NCH
        idxs = [None] * _NCH
        for p in range(_NCH // _PQ):
            cs = range(p * _PQ, (p + 1) * _PQ)
            qx = [qx_v[pl.ds(c * _L, _L)] for c in cs]
            qy = [qy_v[pl.ds(c * _L, _L)] for c in cs]
            qz = [qz_v[pl.ds(c * _L, _L)] for c in cs]
            carry0 = tuple([big] * _PQ + [zero] * _PQ)

            def body(ch, carry, qx=qx, qy=qy, qz=qz):
                mn = list(carry[:_PQ])
                ix = list(carry[_PQ:])
                tbase = ch * _L
                txc = tx_v[pl.ds(tbase, _L)]
                tyc = ty_v[pl.ds(tbase, _L)]
                tzc = tz_v[pl.ds(tbase, _L)]
                for l in range(_L):
                    tx = jnp.full((_L,), txc[l], dtype=jnp.float32)
                    ty = jnp.full((_L,), tyc[l], dtype=jnp.float32)
                    tz = jnp.full((_L,), tzc[l], dtype=jnp.float32)
                    jv = jnp.full((_L,), tbase + l, dtype=jnp.int32)
                    for q in range(_PQ):
                        dx = qx[q] - tx
                        dy = qy[q] - ty
                        dz = qz[q] - tz
                        d2 = dx * dx + dy * dy + dz * dz
                        better = d2 < mn[q]
                        mn[q] = jnp.where(better, d2, mn[q])
                        ix[q] = jnp.where(better, jv, ix[q])
                return tuple(mn + ix)

            carry = lax.fori_loop(0, _N // _L, body, carry0)
            for i, c in enumerate(cs):
                mins[c] = carry[i]
                idxs[c] = carry[_PQ + i]

        for c in range(_NCH):
            sl = pl.ds(c * _L, _L)
            o_d2[sl] = mins[c]
            o_mx[sl] = plsc.load_gather(tx_v, [idxs[c]])
            o_my[sl] = plsc.load_gather(ty_v, [idxs[c]])
            o_mz[sl] = plsc.load_gather(tz_v, [idxs[c]])

        dst = pl.ds(base, _QPW)
        pltpu.sync_copy(o_d2, d2_hbm.at[dst])
        pltpu.sync_copy(o_mx, mx_hbm.at[dst])
        pltpu.sync_copy(o_my, my_hbm.at[dst])
        pltpu.sync_copy(o_mz, mz_hbm.at[dst])

    return knn


_knn = _make_knn_kernel()


def _rigid_transform(p1, p2):
    c1 = jnp.mean(p1, axis=-2, keepdims=True)
    c2 = jnp.mean(p2, axis=-2, keepdims=True)
    q1 = p1 - c1
    q2 = p2 - c2
    H = jnp.einsum('bni,bnj->bij', q1, q2)
    sxx, sxy, sxz = H[:, 0, 0], H[:, 0, 1], H[:, 0, 2]
    syx, syy, syz = H[:, 1, 0], H[:, 1, 1], H[:, 1, 2]
    szx, szy, szz = H[:, 2, 0], H[:, 2, 1], H[:, 2, 2]
    r0 = jnp.stack([sxx + syy + szz, syz - szy, szx - sxz, sxy - syx], -1)
    r1 = jnp.stack([syz - szy, sxx - syy - szz, sxy + syx, szx + sxz], -1)
    r2 = jnp.stack([szx - sxz, sxy + syx, -sxx + syy - szz, syz + szy], -1)
    r3 = jnp.stack([sxy - syx, szx + sxz, syz + szy, -sxx - syy + szz], -1)
    P = jnp.stack([r0, r1, r2, r3], -2)
    c = jnp.sqrt(jnp.sum(P * P, axis=(-2, -1), keepdims=True))
    P = P + c * jnp.eye(4, dtype=P.dtype)[None]
    P = P / jnp.sqrt(jnp.sum(P * P, axis=(-2, -1), keepdims=True))
    for _ in range(8):
        P = jnp.matmul(P, P)
        P = P / jnp.sqrt(jnp.sum(P * P, axis=(-2, -1), keepdims=True))
    diag = jnp.diagonal(P, axis1=-2, axis2=-1)
    col = jnp.argmax(diag, axis=-1)
    q = jnp.take_along_axis(
        P, jnp.broadcast_to(col[:, None, None], (P.shape[0], 4, 1)),
        axis=-1)[..., 0]
    q = q / jnp.linalg.norm(q, axis=-1, keepdims=True)
    w, x, y, z = q[:, 0], q[:, 1], q[:, 2], q[:, 3]
    R = jnp.stack([
        jnp.stack([1 - 2 * (y * y + z * z), 2 * (x * y - w * z),
                   2 * (x * z + w * y)], -1),
        jnp.stack([2 * (x * y + w * z), 1 - 2 * (x * x + z * z),
                   2 * (y * z - w * x)], -1),
        jnp.stack([2 * (x * z - w * y), 2 * (y * z + w * x),
                   1 - 2 * (x * x + y * y)], -1)], -2)
    t = c2[..., 0, :] - jnp.einsum('bij,bj->bi', R, c1[..., 0, :])
    B = p1.shape[0]
    T = jnp.zeros((B, 4, 4), dtype=p1.dtype)
    T = T.at[:, :3, :3].set(R).at[:, :3, 3].set(t).at[:, 3, 3].set(1.0)
    return T


def _apply_se3(T, pts):
    R = T[:, :3, :3]
    t = T[:, :3, 3]
    return jnp.einsum('bij,bnj->bni', R, pts) + t[:, None, :]


def kernel(p1, p2):
    t_x, t_y, t_z = p2[0, :, 0], p2[0, :, 1], p2[0, :, 2]

    err0 = jnp.zeros((1,), dtype=p1.dtype)
    done0 = jnp.array(False)
    it0 = jnp.array(0, dtype=jnp.int32)

    def cond_fn(carry):
        it, err, done, temppc = carry
        return jnp.logical_and(it <= _STEPLIM, jnp.logical_not(done))

    def body_fn(carry):
        it, err, done, temppc = carry
        it = it + 1
        d2, mx, my, mz = _knn(temppc[0, :, 0], temppc[0, :, 1],
                              temppc[0, :, 2], t_x, t_y, t_z)
        vals = jnp.sqrt(d2)
        errnew = jnp.sum(vals).reshape(1)
        matched = jnp.stack([mx, my, mz], axis=-1)[None]
        T = _rigid_transform(temppc, matched)
        temppc = _apply_se3(T, temppc)
        converged = jnp.abs(err - errnew)[0] < _TOL
        err = jnp.where(converged, err, errnew)
        done = converged
        return (it, err, done, temppc)

    it_f, err_f, done_f, temppc = lax.while_loop(
        cond_fn, body_fn, (it0, err0, done0, p1))
    return _rigid_transform(p1, temppc)


# speedup vs baseline: 1.8317x; 1.8317x over previous
import functools

import jax
import jax.numpy as jnp
from jax import lax
from jax.experimental import pallas as pl
from jax.experimental.pallas import tpu as pltpu
from jax.experimental.pallas import tpu_sc as plsc

_N = 2048
_L = 16
_NC = 2
_NS = 16
_NW = _NC * _NS
_QPW = _N // _NW
_NCH = _QPW // _L
_NPART = 16

_STEPLIM = 5
_TOL = 1e-4


def _make_knn_kernel():
    mesh = plsc.VectorSubcoreMesh(
        core_axis_name="c", subcore_axis_name="s",
        num_cores=_NC, num_subcores=_NS)

    out_n = jax.ShapeDtypeStruct((_N,), jnp.float32)

    @functools.partial(
        pl.kernel,
        out_type=(out_n, out_n, out_n, out_n,
                  jax.ShapeDtypeStruct((_NW * _NPART * _L,), jnp.float32)),
        mesh=mesh,
        compiler_params=pltpu.CompilerParams(needs_layout_passes=False),
        scratch_types=(
            pltpu.VMEM((_N,), jnp.float32),
            pltpu.VMEM((_N,), jnp.float32),
            pltpu.VMEM((_N,), jnp.float32),
            pltpu.VMEM((_QPW,), jnp.float32),
            pltpu.VMEM((_QPW,), jnp.float32),
            pltpu.VMEM((_QPW,), jnp.float32),
            pltpu.VMEM((_L,), jnp.float32),
            pltpu.VMEM((_QPW,), jnp.float32),
            pltpu.VMEM((_NPART * _L,), jnp.float32),
        ),
    )
    def knn(qx_hbm, qy_hbm, qz_hbm, tx_hbm, ty_hbm, tz_hbm, rt_hbm,
            nqx_hbm, nqy_hbm, nqz_hbm, d2_hbm, part_hbm,
            tx_v, ty_v, tz_v, qx_v, qy_v, qz_v, rt_v, o_d2, o_part):
        wid = lax.axis_index("s") * _NC + lax.axis_index("c")
        base = wid * _QPW

        pltpu.sync_copy(tx_hbm, tx_v)
        pltpu.sync_copy(ty_hbm, ty_v)
        pltpu.sync_copy(tz_hbm, tz_v)
        pltpu.sync_copy(qx_hbm.at[pl.ds(base, _QPW)], qx_v)
        pltpu.sync_copy(qy_hbm.at[pl.ds(base, _QPW)], qy_v)
        pltpu.sync_copy(qz_hbm.at[pl.ds(base, _QPW)], qz_v)
        pltpu.sync_copy(rt_hbm, rt_v)

        rtc = rt_v[pl.ds(0, _L)]
        rb = [jnp.full((_L,), rtc[i], dtype=jnp.float32) for i in range(12)]
        (r00, r01, r02, r10, r11, r12, r20, r21, r22, t0, t1, t2) = rb

        for c in range(_NCH):
            sl = pl.ds(c * _L, _L)
            ox, oy, oz = qx_v[sl], qy_v[sl], qz_v[sl]
            qx_v[sl] = r00 * ox + r01 * oy + r02 * oz + t0
            qy_v[sl] = r10 * ox + r11 * oy + r12 * oz + t1
            qz_v[sl] = r20 * ox + r21 * oy + r22 * oz + t2

        big = jnp.full((_L,), jnp.inf, dtype=jnp.float32)
        zero = jnp.zeros((_L,), dtype=jnp.int32)

        mins = [None] * _NCH
        idxs = [None] * _NCH
        for c in range(_NCH):
            qx = qx_v[pl.ds(c * _L, _L)]
            qy = qy_v[pl.ds(c * _L, _L)]
            qz = qz_v[pl.ds(c * _L, _L)]

            def body(ch, carry, qx=qx, qy=qy, qz=qz):
                mn, ix = carry
                tbase = ch * _L
                txc = tx_v[pl.ds(tbase, _L)]
                tyc = ty_v[pl.ds(tbase, _L)]
                tzc = tz_v[pl.ds(tbase, _L)]
                for l in range(_L):
                    tx = jnp.full((_L,), txc[l], dtype=jnp.float32)
                    ty = jnp.full((_L,), tyc[l], dtype=jnp.float32)
                    tz = jnp.full((_L,), tzc[l], dtype=jnp.float32)
                    jv = jnp.full((_L,), tbase + l, dtype=jnp.int32)
                    dx = qx - tx
                    dy = qy - ty
                    dz = qz - tz
                    d2 = dx * dx + dy * dy + dz * dz
                    better = d2 < mn
                    mn = jnp.where(better, d2, mn)
                    ix = jnp.where(better, jv, ix)
                return (mn, ix)

            mins[c], idxs[c] = lax.fori_loop(0, _N // _L, body, (big, zero))

        acc = [jnp.zeros((_L,), dtype=jnp.float32) for _ in range(15)]
        for c in range(_NCH):
            sl = pl.ds(c * _L, _L)
            o_d2[sl] = mins[c]
            qx = qx_v[sl]
            qy = qy_v[sl]
            qz = qz_v[sl]
            mx = plsc.load_gather(tx_v, [idxs[c]])
            my = plsc.load_gather(ty_v, [idxs[c]])
            mz = plsc.load_gather(tz_v, [idxs[c]])
            qs = (qx, qy, qz)
            ms = (mx, my, mz)
            for i in range(3):
                for j in range(3):
                    acc[3 * i + j] = acc[3 * i + j] + qs[i] * ms[j]
            for j in range(3):
                acc[9 + j] = acc[9 + j] + ms[j]
                acc[12 + j] = acc[12 + j] + qs[j]

        for k in range(15):
            o_part[pl.ds(k * _L, _L)] = acc[k]
        o_part[pl.ds(15 * _L, _L)] = jnp.zeros((_L,), dtype=jnp.float32)

        dst = pl.ds(base, _QPW)
        pltpu.sync_copy(qx_v, nqx_hbm.at[dst])
        pltpu.sync_copy(qy_v, nqy_hbm.at[dst])
        pltpu.sync_copy(qz_v, nqz_hbm.at[dst])
        pltpu.sync_copy(o_d2, d2_hbm.at[dst])
        pltpu.sync_copy(o_part, part_hbm.at[pl.ds(wid * _NPART * _L,
                                                  _NPART * _L)])

    return knn


_knn = _make_knn_kernel()


def _quat_R_from_H(h):
    sxx, sxy, sxz = h[0][0], h[0][1], h[0][2]
    syx, syy, syz = h[1][0], h[1][1], h[1][2]
    szx, szy, szz = h[2][0], h[2][1], h[2][2]
    n_elems = [sxx + syy + szz, syz - szy, szx - sxz, sxy - syx,
               syz - szy, sxx - syy - szz, sxy + syx, szx + sxz,
               szx - sxz, sxy + syx, -sxx + syy - szz, syz + szy,
               sxy - syx, szx + sxz, syz + szy, -sxx - syy + szz]
    rows = []
    for i in range(4):
        rows.append(jnp.concatenate(
            [jnp.full((1, 1), n_elems[4 * i + j], dtype=jnp.float32)
             for j in range(4)], axis=1))
    P = jnp.concatenate(rows, axis=0)
    fro = jnp.sqrt(jnp.sum(P * P))
    eye4 = jnp.where(
        lax.broadcasted_iota(jnp.int32, (4, 4), 0)
        == lax.broadcasted_iota(jnp.int32, (4, 4), 1), 1.0, 0.0)
    P = P + fro * eye4
    P = P / jnp.max(jnp.abs(P))
    for _ in range(8):
        P = jnp.dot(P, P, preferred_element_type=jnp.float32)
        P = P / jnp.max(jnp.abs(P))
    diag = jnp.sum(P * eye4, axis=1)
    col = jnp.argmax(diag, axis=0)
    onehot = jnp.where(lax.iota(jnp.int32, 4) == col, 1.0, 0.0)
    q = jnp.sum(P * onehot[None, :], axis=1)
    w, x, y, z = q[0], q[1], q[2], q[3]
    nn = w * w + x * x + y * y + z * z
    inv = 1.0 / nn
    return [
        [(nn - 2 * (y * y + z * z)) * inv, 2 * (x * y - w * z) * inv,
         2 * (x * z + w * y) * inv],
        [2 * (x * y + w * z) * inv, (nn - 2 * (x * x + z * z)) * inv,
         2 * (y * z - w * x) * inv],
        [2 * (x * z - w * y) * inv, 2 * (y * z + w * x) * inv,
         (nn - 2 * (x * x + y * y)) * inv],
    ]


def _solve_body(part_ref, d2_ref, err_ref, rt_ref, errnew_ref, done_ref):
    part = part_ref[...]
    colsum = jnp.sum(part, axis=0, keepdims=True)
    gidx = lax.broadcasted_iota(jnp.int32, (_NPART * _L, _NPART), 0) // _L
    sel = jnp.where(
        gidx == lax.broadcasted_iota(jnp.int32, (_NPART * _L, _NPART), 1),
        1.0, 0.0)
    tot = jnp.dot(colsum, sel, preferred_element_type=jnp.float32)
    s = [tot[0, k] for k in range(15)]
    n_inv = 1.0 / _N
    h = [[s[3 * i + j] - s[12 + i] * s[9 + j] * n_inv for j in range(3)]
         for i in range(3)]
    R = _quat_R_from_H(h)
    c1 = [s[12 + i] * n_inv for i in range(3)]
    c2 = [s[9 + i] * n_inv for i in range(3)]
    t = [c2[i] - (R[i][0] * c1[0] + R[i][1] * c1[1] + R[i][2] * c1[2])
         for i in range(3)]
    flat = [R[0][0], R[0][1], R[0][2],
            R[1][0], R[1][1], R[1][2],
            R[2][0], R[2][1], R[2][2],
            t[0], t[1], t[2],
            jnp.float32(0.0), jnp.float32(0.0), jnp.float32(0.0),
            jnp.float32(0.0)]
    rt_ref[...] = jnp.concatenate(
        [jnp.full((1, 1), v, dtype=jnp.float32) for v in flat], axis=1)
    errnew = jnp.sum(jnp.sqrt(d2_ref[...]))
    errnew_ref[...] = jnp.full((1, 1), errnew, dtype=jnp.float32)
    done_ref[...] = jnp.where(
        jnp.abs(err_ref[0, 0] - errnew) < _TOL,
        jnp.full((1, 1), 1, dtype=jnp.int32),
        jnp.full((1, 1), 0, dtype=jnp.int32))


_solve = pl.pallas_call(
    _solve_body,
    out_shape=(
        jax.ShapeDtypeStruct((1, 16), jnp.float32),
        jax.ShapeDtypeStruct((1, 1), jnp.float32),
        jax.ShapeDtypeStruct((1, 1), jnp.int32),
    ),
)


def _solve_final(p1, temppc):
    s_mat = jnp.einsum('ni,nj->ij', p1[0], temppc[0])
    sq = jnp.sum(p1[0], axis=0)
    sm = jnp.sum(temppc[0], axis=0)
    h = [[s_mat[i, j] - sq[i] * sm[j] / _N for j in range(3)]
         for i in range(3)]
    sxx, sxy, sxz = h[0][0], h[0][1], h[0][2]
    syx, syy, syz = h[1][0], h[1][1], h[1][2]
    szx, szy, szz = h[2][0], h[2][1], h[2][2]
    r0 = jnp.stack([sxx + syy + szz, syz - szy, szx - sxz, sxy - syx])
    r1 = jnp.stack([syz - szy, sxx - syy - szz, sxy + syx, szx + sxz])
    r2 = jnp.stack([szx - sxz, sxy + syx, -sxx + syy - szz, syz + szy])
    r3 = jnp.stack([sxy - syx, szx + sxz, syz + szy, -sxx - syy + szz])
    P = jnp.stack([r0, r1, r2, r3])
    P = P + jnp.sqrt(jnp.sum(P * P)) * jnp.eye(4, dtype=P.dtype)
    P = P / jnp.max(jnp.abs(P))
    for _ in range(8):
        P = P @ P
        P = P / jnp.max(jnp.abs(P))
    diag = jnp.diag(P)
    col = jnp.argmax(diag)
    q = P @ (jnp.arange(4) == col).astype(P.dtype)
    w, x, y, z = q[0], q[1], q[2], q[3]
    nn = w * w + x * x + y * y + z * z
    R = jnp.stack([
        jnp.stack([nn - 2 * (y * y + z * z), 2 * (x * y - w * z),
                   2 * (x * z + w * y)]),
        jnp.stack([2 * (x * y + w * z), nn - 2 * (x * x + z * z),
                   2 * (y * z - w * x)]),
        jnp.stack([2 * (x * z - w * y), 2 * (y * z + w * x),
                   nn - 2 * (x * x + y * y)])]) / nn
    c1 = sq / _N
    c2 = sm / _N
    t = c2 - R @ c1
    T = jnp.zeros((1, 4, 4), dtype=jnp.float32)
    T = T.at[0, :3, :3].set(R).at[0, :3, 3].set(t).at[0, 3, 3].set(1.0)
    return T


def kernel(p1, p2):
    t_x, t_y, t_z = p2[0, :, 0], p2[0, :, 1], p2[0, :, 2]
    q_x, q_y, q_z = p1[0, :, 0], p1[0, :, 1], p1[0, :, 2]

    rt0 = jnp.array([1, 0, 0, 0, 1, 0, 0, 0, 1, 0, 0, 0, 0, 0, 0, 0],
                    dtype=jnp.float32)
    err0 = jnp.zeros((1,), dtype=jnp.float32)
    done0 = jnp.array(False)
    it0 = jnp.array(0, dtype=jnp.int32)

    def cond_fn(carry):
        it, err, done, qx, qy, qz, rt = carry
        return jnp.logical_and(it <= _STEPLIM, jnp.logical_not(done))

    def body_fn(carry):
        it, err, done, qx, qy, qz, rt = carry
        it = it + 1
        nqx, nqy, nqz, d2, part = _knn(qx, qy, qz, t_x, t_y, t_z, rt)
        rt_n, errnew, conv = _solve(
            part.reshape(_NW, _NPART * _L), d2.reshape(16, 128),
            err.reshape(1, 1))
        errnew = errnew.reshape(1)
        converged = conv[0, 0] == 1
        err = jnp.where(converged, err, errnew)
        return (it, err, converged, nqx, nqy, nqz, rt_n.reshape(16))

    it_f, err_f, done_f, qx, qy, qz, rt = lax.while_loop(
        cond_fn, body_fn, (it0, err0, done0, q_x, q_y, q_z, rt0))

    fx = rt[0] * qx + rt[1] * qy + rt[2] * qz + rt[9]
    fy = rt[3] * qx + rt[4] * qy + rt[5] * qz + rt[10]
    fz = rt[6] * qx + rt[7] * qy + rt[8] * qz + rt[11]
    temppc = jnp.stack([fx, fy, fz], axis=-1)[None]
    return _solve_final(p1, temppc)
